# baseline (speedup 1.0000x reference)
import functools

import jax
import jax.numpy as jnp
from jax import lax
from jax.experimental import pallas as pl
from jax.experimental.pallas import tpu as pltpu
from jax.experimental.pallas import tpu_sc as plsc

E_SMOOTH = 0.1
BLK = 2048

NC = 2
NS = 16
NW = NC * NS
GCHUNK = 128


def _loss_block_kernel(x_ref, out_ref, *, n_cols):
    xb = x_ref[...]
    m = jnp.max(xb, axis=1)
    s = jnp.sum(jnp.exp(xb - m[:, None]), axis=1)
    rs = jnp.sum(xb, axis=1)
    partial = jnp.sum(m + jnp.log(s) - (E_SMOOTH / n_cols) * rs).reshape(1, 1)

    @pl.when(pl.program_id(0) == 0)
    def _():
        out_ref[...] = jnp.zeros((1, 1), jnp.float32)

    out_ref[...] += partial


def _dense_loss_part(x):
    B, C = x.shape
    out = pl.pallas_call(
        functools.partial(_loss_block_kernel, n_cols=C),
        grid=(B // BLK,),
        in_specs=[pl.BlockSpec((BLK, C), lambda i: (i, 0))],
        out_specs=pl.BlockSpec((1, 1), lambda i: (0, 0)),
        out_shape=jax.ShapeDtypeStruct((1, 1), jnp.float32),
        compiler_params=pltpu.CompilerParams(
            dimension_semantics=("arbitrary",),
        ),
    )(x)
    return out[0, 0]


def _sc_gather_sum(x_flat, flat_idx):
    B = flat_idx.shape[0]
    per_w = B // NW
    n_chunks = per_w // GCHUNK
    mesh = plsc.VectorSubcoreMesh(core_axis_name="c", subcore_axis_name="s")

    @functools.partial(
        pl.kernel,
        mesh=mesh,
        out_type=jax.ShapeDtypeStruct((NW, 16), jnp.float32),
        scratch_types=[
            pltpu.VMEM((per_w,), jnp.int32),
            pltpu.VMEM((per_w,), jnp.float32),
            pltpu.VMEM((16,), jnp.float32),
            pltpu.SemaphoreType.DMA,
        ],
    )
    def gather_kernel(x_hbm, idx_hbm, out_hbm, idx_v, vals_v, acc_v, sem):
        wid = lax.axis_index("s") * NC + lax.axis_index("c")
        base = wid * per_w
        pltpu.sync_copy(idx_hbm.at[pl.ds(base, per_w)], idx_v)
        copies = []
        for k in range(n_chunks):
            sl = pl.ds(k * GCHUNK, GCHUNK)
            copies.append(
                pltpu.async_copy(x_hbm.at[idx_v.at[sl]], vals_v.at[sl], sem)
            )
        for cp in copies:
            cp.wait()
        acc = jnp.zeros((16,), jnp.float32)
        for j in range(per_w // 16):
            acc = acc + vals_v[pl.ds(j * 16, 16)]
        acc_v[...] = acc
        pltpu.sync_copy(acc_v, out_hbm.at[wid])

    return gather_kernel(x_flat, flat_idx)


def kernel(x, target):
    B, C = x.shape
    target = target.astype(jnp.int32)
    flat_idx = jnp.arange(B, dtype=jnp.int32) * C + target
    dense = _dense_loss_part(x)
    sc_parts = _sc_gather_sum(x.reshape(-1), flat_idx)
    xt_sum = jnp.sum(sc_parts)
    return (dense - (1.0 - E_SMOOTH) * xt_sum) / B


# speedup vs baseline: 2.0106x; 2.0106x over previous
import functools

import jax
import jax.numpy as jnp
from jax import lax
from jax.experimental import pallas as pl
from jax.experimental.pallas import tpu as pltpu

E_SMOOTH = 0.1
BLK = 2048


def _loss_block_kernel(x_ref, t_ref, out_ref, *, n_cols):
    xb = x_ref[...]
    tb = t_ref[0, 0, :]
    m = jnp.max(xb, axis=1)
    s = jnp.sum(jnp.exp(xb - m[:, None]), axis=1)
    rs = jnp.sum(xb, axis=1)
    cols = lax.broadcasted_iota(jnp.int32, xb.shape, 1)
    xt = jnp.sum(jnp.where(cols == tb[:, None], xb, 0.0), axis=1)
    partial = jnp.sum(
        m + jnp.log(s) - (E_SMOOTH / n_cols) * rs - (1.0 - E_SMOOTH) * xt
    ).reshape(1, 1)

    @pl.when(pl.program_id(0) == 0)
    def _():
        out_ref[...] = jnp.zeros((1, 1), jnp.float32)

    out_ref[...] += partial


def kernel(x, target):
    B, C = x.shape
    target = target.astype(jnp.int32)
    n_blocks = B // BLK
    t3 = target.reshape(n_blocks, 1, BLK)

    out = pl.pallas_call(
        functools.partial(_loss_block_kernel, n_cols=C),
        grid=(n_blocks,),
        in_specs=[
            pl.BlockSpec((BLK, C), lambda i: (i, 0)),
            pl.BlockSpec((1, 1, BLK), lambda i: (i, 0, 0)),
        ],
        out_specs=pl.BlockSpec((1, 1), lambda i: (0, 0)),
        out_shape=jax.ShapeDtypeStruct((1, 1), jnp.float32),
        compiler_params=pltpu.CompilerParams(
            dimension_semantics=("arbitrary",),
        ),
    )(x, t3)
    return out[0, 0] / B


# speedup vs baseline: 2.0297x; 1.0095x over previous
import functools

import jax
import jax.numpy as jnp
from jax import lax
from jax.experimental import pallas as pl
from jax.experimental.pallas import tpu as pltpu

E_SMOOTH = 0.1
BLK = 2048


def _loss_block_kernel(x_ref, t_ref, out_ref, *, n_cols):
    xb = x_ref[...]
    tb = t_ref[0, 0, :]
    m = jnp.max(xb, axis=1)
    s = jnp.sum(jnp.exp(xb - m[:, None]), axis=1)
    k = (1.0 - E_SMOOTH) * n_cols / E_SMOOTH
    cols = lax.broadcasted_iota(jnp.int32, xb.shape, 1)
    comb = jnp.sum(
        jnp.where(cols == tb[:, None], xb * (1.0 + k), xb), axis=1
    )
    partial = jnp.sum(
        m + jnp.log(s) - (E_SMOOTH / n_cols) * comb
    ).reshape(1, 1)

    @pl.when(pl.program_id(0) == 0)
    def _():
        out_ref[...] = jnp.zeros((1, 1), jnp.float32)

    out_ref[...] += partial


def kernel(x, target):
    B, C = x.shape
    target = target.astype(jnp.int32)
    n_blocks = B // BLK
    t3 = target.reshape(n_blocks, 1, BLK)

    out = pl.pallas_call(
        functools.partial(_loss_block_kernel, n_cols=C),
        grid=(n_blocks,),
        in_specs=[
            pl.BlockSpec((BLK, C), lambda i: (i, 0)),
            pl.BlockSpec((1, 1, BLK), lambda i: (i, 0, 0)),
        ],
        out_specs=pl.BlockSpec((1, 1), lambda i: (0, 0)),
        out_shape=jax.ShapeDtypeStruct((1, 1), jnp.float32),
        compiler_params=pltpu.CompilerParams(
            dimension_semantics=("arbitrary",),
        ),
    )(x, t3)
    return out[0, 0] / B


# speedup vs baseline: 2.0343x; 1.0022x over previous
import functools

import jax
import jax.numpy as jnp
from jax import lax
from jax.experimental import pallas as pl
from jax.experimental.pallas import tpu as pltpu

E_SMOOTH = 0.1
BLK = 2048


def _loss_block_kernel(x_ref, t_ref, out_ref, *, n_cols):
    xb = x_ref[...]
    tb = t_ref[0, 0, :]
    m = jnp.max(xb, axis=1)
    ones = jnp.ones((xb.shape[1],), jnp.float32)
    s = jnp.dot(jnp.exp(xb - m[:, None]), ones,
                preferred_element_type=jnp.float32)
    k = (1.0 - E_SMOOTH) * n_cols / E_SMOOTH
    cols = lax.broadcasted_iota(jnp.int32, xb.shape, 1)
    comb = jnp.dot(
        jnp.where(cols == tb[:, None], xb * (1.0 + k), xb), ones,
        preferred_element_type=jnp.float32,
    )
    partial = jnp.sum(
        m + jnp.log(s) - (E_SMOOTH / n_cols) * comb
    ).reshape(1, 1)

    @pl.when(pl.program_id(0) == 0)
    def _():
        out_ref[...] = jnp.zeros((1, 1), jnp.float32)

    out_ref[...] += partial


def kernel(x, target):
    B, C = x.shape
    target = target.astype(jnp.int32)
    n_blocks = B // BLK
    t3 = target.reshape(n_blocks, 1, BLK)

    out = pl.pallas_call(
        functools.partial(_loss_block_kernel, n_cols=C),
        grid=(n_blocks,),
        in_specs=[
            pl.BlockSpec((BLK, C), lambda i: (i, 0)),
            pl.BlockSpec((1, 1, BLK), lambda i: (i, 0, 0)),
        ],
        out_specs=pl.BlockSpec((1, 1), lambda i: (0, 0)),
        out_shape=jax.ShapeDtypeStruct((1, 1), jnp.float32),
        compiler_params=pltpu.CompilerParams(
            dimension_semantics=("arbitrary",),
        ),
    )(x, t3)
    return out[0, 0] / B
